# baseline (speedup 1.0000x reference)
import jax
import jax.numpy as jnp
from jax.experimental import pallas as pl
from jax.experimental.pallas import tpu as pltpu

B, C, L = 16, 256, 1024
DIM_U, Q_EMB = 64, 64
C2 = 2 * C
EPS = 1e-5


def _mix_body(u_ref, wt_ref, out_ref):
    out_ref[...] = jnp.dot(u_ref[...], wt_ref[...],
                           preferred_element_type=jnp.float32)


def _main_body(x_ref, m_ref, e_ref, t_ref, v_ref, b_ref, out_ref):
    xb = x_ref[0]
    mu = jnp.mean(xb, axis=1, keepdims=True)
    xc = xb - mu
    var = jnp.mean(xc * xc, axis=1, keepdims=True)
    nx = xc * jax.lax.rsqrt(var + EPS)
    style = jnp.dot(m_ref[0], e_ref[0], preferred_element_type=jnp.float32)
    params = style + v_ref[...] * t_ref[0] + b_ref[...]
    out_ref[0] = (1.0 + params[:C]) * nx + params[C:]


def kernel(x, u_i, e_qid, t, W, V, bias):
    wt = W.reshape(C2, DIM_U, Q_EMB).transpose(1, 0, 2).reshape(DIM_U, C2 * Q_EMB)

    mflat = pl.pallas_call(
        _mix_body,
        out_shape=jax.ShapeDtypeStruct((B, C2 * Q_EMB), jnp.float32),
        name="adain_mix",
    )(u_i, wt)
    m3 = mflat.reshape(B, C2, Q_EMB)

    out = pl.pallas_call(
        _main_body,
        out_shape=jax.ShapeDtypeStruct((B, C, L), jnp.float32),
        grid=(B,),
        in_specs=[
            pl.BlockSpec((1, C, L), lambda b: (b, 0, 0)),
            pl.BlockSpec((1, C2, Q_EMB), lambda b: (b, 0, 0)),
            pl.BlockSpec((1, Q_EMB, L), lambda b: (b, 0, 0)),
            pl.BlockSpec((1, 1, L), lambda b: (b, 0, 0)),
            pl.BlockSpec((C2, 1), lambda b: (0, 0)),
            pl.BlockSpec((C2, 1), lambda b: (0, 0)),
        ],
        out_specs=pl.BlockSpec((1, C, L), lambda b: (b, 0, 0)),
        compiler_params=pltpu.CompilerParams(
            dimension_semantics=("parallel",),
        ),
        name="adain_main",
    )(x, m3, e_qid, t, V, bias.reshape(C2, 1))
    return out


# speedup vs baseline: 2.6928x; 2.6928x over previous
import jax
import jax.numpy as jnp
from jax.experimental import pallas as pl
from jax.experimental.pallas import tpu as pltpu

B, C, L = 16, 256, 1024
DIM_U, Q_EMB = 64, 64
INTER = DIM_U * Q_EMB
C2 = 2 * C
EPS = 1e-5


def _body(u_smem, x_ref, e_ref, t_ref, w_ref, v_ref, b2_ref, out_ref):
    i = pl.program_id(0)
    xb = x_ref[0]
    mu = jnp.sum(xb, axis=1, keepdims=True) * (1.0 / L)
    var = jnp.sum(xb * xb, axis=1, keepdims=True) * (1.0 / L) - mu * mu
    rstd = jax.lax.rsqrt(var + EPS)
    lane = jax.lax.broadcasted_iota(jnp.int32, (1, 128), 1)
    acc = jnp.zeros((C2, 128), jnp.float32)
    for k in range(DIM_U // 2):
        s0 = u_smem[i, 2 * k]
        s1 = u_smem[i, 2 * k + 1]
        m = jnp.where(lane < Q_EMB, s0, s1)
        acc = acc + w_ref[:, 128 * k:128 * (k + 1)] * m
    lhs = jnp.concatenate([acc, v_ref[...], b2_ref[...]], axis=1)
    rhs = jnp.concatenate(
        [e_ref[0], e_ref[0], t_ref[0], jnp.ones((1, L), jnp.float32)], axis=0)
    params = jnp.dot(lhs, rhs, preferred_element_type=jnp.float32)
    nx = (xb - mu) * rstd
    out_ref[0] = params[:C] * nx + params[C:]


def kernel(x, u_i, e_qid, t, W, V, bias):
    return pl.pallas_call(
        _body,
        out_shape=jax.ShapeDtypeStruct((B, C, L), jnp.float32),
        grid=(B,),
        in_specs=[
            pl.BlockSpec(memory_space=pltpu.SMEM),
            pl.BlockSpec((1, C, L), lambda b: (b, 0, 0)),
            pl.BlockSpec((1, Q_EMB, L), lambda b: (b, 0, 0)),
            pl.BlockSpec((1, 1, L), lambda b: (b, 0, 0)),
            pl.BlockSpec((C2, INTER), lambda b: (0, 0)),
            pl.BlockSpec((C2, 1), lambda b: (0, 0)),
            pl.BlockSpec((C2, 1), lambda b: (0, 0)),
        ],
        out_specs=pl.BlockSpec((1, C, L), lambda b: (b, 0, 0)),
        compiler_params=pltpu.CompilerParams(
            dimension_semantics=("parallel",),
        ),
        name="adain_fused",
    )(u_i, x, e_qid, t, W, V,
      bias[:, None] + (jnp.arange(C2) < C).astype(jnp.float32)[:, None])


# speedup vs baseline: 3.0730x; 1.1412x over previous
import jax
import jax.numpy as jnp
from jax.experimental import pallas as pl
from jax.experimental.pallas import tpu as pltpu

B, C, L = 16, 256, 1024
DIM_U, Q_EMB = 64, 64
INTER = DIM_U * Q_EMB
C2 = 2 * C
EPS = 1e-5
BB = 2
GRID = B // BB


def _body(u_smem, x_ref, e_ref, w_ref, t_any, vb_any, out_ref,
          t_vmem, vb_vmem, sem_t, sem_vb):
    i = pl.program_id(0)

    @pl.when(i == 0)
    def _():
        ct = pltpu.make_async_copy(t_any, t_vmem, sem_t)
        cv = pltpu.make_async_copy(vb_any, vb_vmem, sem_vb)
        ct.start()
        cv.start()
        ct.wait()
        cv.wait()

    lane = jax.lax.broadcasted_iota(jnp.int32, (1, 128), 1)
    ones_row = jnp.ones((1, L), jnp.float32)
    for j in range(BB):
        b = BB * i + j
        xb = x_ref[j]
        mu = jnp.sum(xb, axis=1, keepdims=True) * (1.0 / L)
        var = jnp.sum(xb * xb, axis=1, keepdims=True) * (1.0 / L) - mu * mu
        rstd = jax.lax.rsqrt(var + EPS)
        acc = jnp.zeros((C2, 128), jnp.float32)
        for k in range(DIM_U // 2):
            s0 = u_smem[b, 2 * k]
            s1 = u_smem[b, 2 * k + 1]
            m = jnp.where(lane < Q_EMB, s0, s1)
            acc = acc + w_ref[:, 128 * k:128 * (k + 1)] * m
        lhs = jnp.concatenate([acc, vb_vmem[...]], axis=1)
        rhs = jnp.concatenate(
            [e_ref[j], e_ref[j], t_vmem[b], ones_row], axis=0)
        params = jnp.dot(lhs, rhs, preferred_element_type=jnp.float32)
        nx = (xb - mu) * rstd
        out_ref[j] = params[:C] * nx + params[C:]


def kernel(x, u_i, e_qid, t, W, V, bias):
    vb = jnp.concatenate(
        [V, bias[:, None] + (jnp.arange(C2) < C).astype(jnp.float32)[:, None]],
        axis=1)
    return pl.pallas_call(
        _body,
        out_shape=jax.ShapeDtypeStruct((B, C, L), jnp.float32),
        grid=(GRID,),
        in_specs=[
            pl.BlockSpec(memory_space=pltpu.SMEM),
            pl.BlockSpec((BB, C, L), lambda g: (g, 0, 0)),
            pl.BlockSpec((BB, Q_EMB, L), lambda g: (g, 0, 0)),
            pl.BlockSpec((C2, INTER), lambda g: (0, 0)),
            pl.BlockSpec(memory_space=pl.ANY),
            pl.BlockSpec(memory_space=pl.ANY),
        ],
        out_specs=pl.BlockSpec((BB, C, L), lambda g: (g, 0, 0)),
        scratch_shapes=[
            pltpu.VMEM((B, 1, L), jnp.float32),
            pltpu.VMEM((C2, 2), jnp.float32),
            pltpu.SemaphoreType.DMA,
            pltpu.SemaphoreType.DMA,
        ],
        compiler_params=pltpu.CompilerParams(
            dimension_semantics=("arbitrary",),
        ),
        name="adain_fused",
    )(u_i, x, e_qid, W, t, vb)


# speedup vs baseline: 3.1681x; 1.0310x over previous
import jax
import jax.numpy as jnp
from jax.experimental import pallas as pl
from jax.experimental.pallas import tpu as pltpu

B, C, L = 16, 256, 1024
DIM_U, Q_EMB = 64, 64
INTER = DIM_U * Q_EMB
C2 = 2 * C
EPS = 1e-5
BB = 2
NCH = B // BB


def _body(u_smem, x_any, e_any, w_any, t_any, vb_any, out_any,
          xbuf, ebuf, obuf, wbuf, tbuf, vbbuf,
          xsem, esem, osem, wsem, tsem, vbsem):
    def copy_in(i, slot):
        pltpu.make_async_copy(
            x_any.at[pl.ds(BB * i, BB)], xbuf.at[slot], xsem.at[slot]).start()
        pltpu.make_async_copy(
            e_any.at[pl.ds(BB * i, BB)], ebuf.at[slot], esem.at[slot]).start()

    pltpu.make_async_copy(w_any, wbuf, wsem).start()
    pltpu.make_async_copy(t_any, tbuf, tsem).start()
    pltpu.make_async_copy(vb_any, vbbuf, vbsem).start()
    copy_in(0, 0)
    pltpu.make_async_copy(w_any, wbuf, wsem).wait()
    pltpu.make_async_copy(t_any, tbuf, tsem).wait()
    pltpu.make_async_copy(vb_any, vbbuf, vbsem).wait()

    lane = jax.lax.broadcasted_iota(jnp.int32, (1, 128), 1)
    ones_row = jnp.ones((1, L), jnp.float32)
    for i in range(NCH):
        slot = i % 2
        if i < NCH - 1:
            copy_in(i + 1, 1 - slot)
        pltpu.make_async_copy(
            x_any.at[pl.ds(BB * i, BB)], xbuf.at[slot], xsem.at[slot]).wait()
        pltpu.make_async_copy(
            e_any.at[pl.ds(BB * i, BB)], ebuf.at[slot], esem.at[slot]).wait()
        if i >= 2:
            pltpu.make_async_copy(
                obuf.at[slot], out_any.at[pl.ds(BB * (i - 2), BB)],
                osem.at[slot]).wait()
        for j in range(BB):
            b = BB * i + j
            xb = xbuf[slot, j]
            mu = jnp.sum(xb, axis=1, keepdims=True) * (1.0 / L)
            var = jnp.sum(xb * xb, axis=1, keepdims=True) * (1.0 / L) - mu * mu
            rstd = jax.lax.rsqrt(var + EPS)
            acc = jnp.zeros((C2, 128), jnp.float32)
            for k in range(DIM_U // 2):
                s0 = u_smem[b, 2 * k]
                s1 = u_smem[b, 2 * k + 1]
                m = jnp.where(lane < Q_EMB, s0, s1)
                acc = acc + wbuf[:, 128 * k:128 * (k + 1)] * m
            lhs = jnp.concatenate([acc, vbbuf[...]], axis=1)
            rhs = jnp.concatenate(
                [ebuf[slot, j], ebuf[slot, j], tbuf[b], ones_row], axis=0)
            params = jnp.dot(lhs, rhs, preferred_element_type=jnp.float32)
            nx = (xb - mu) * rstd
            obuf[slot, j] = params[:C] * nx + params[C:]
        pltpu.make_async_copy(
            obuf.at[slot], out_any.at[pl.ds(BB * i, BB)], osem.at[slot]).start()
    for i in range(NCH - 2, NCH):
        slot = i % 2
        pltpu.make_async_copy(
            obuf.at[slot], out_any.at[pl.ds(BB * i, BB)], osem.at[slot]).wait()


def kernel(x, u_i, e_qid, t, W, V, bias):
    vb = jnp.concatenate(
        [V, bias[:, None] + (jnp.arange(C2) < C).astype(jnp.float32)[:, None]],
        axis=1)
    return pl.pallas_call(
        _body,
        out_shape=jax.ShapeDtypeStruct((B, C, L), jnp.float32),
        in_specs=[
            pl.BlockSpec(memory_space=pltpu.SMEM),
            pl.BlockSpec(memory_space=pl.ANY),
            pl.BlockSpec(memory_space=pl.ANY),
            pl.BlockSpec(memory_space=pl.ANY),
            pl.BlockSpec(memory_space=pl.ANY),
            pl.BlockSpec(memory_space=pl.ANY),
        ],
        out_specs=pl.BlockSpec(memory_space=pl.ANY),
        scratch_shapes=[
            pltpu.VMEM((2, BB, C, L), jnp.float32),
            pltpu.VMEM((2, BB, Q_EMB, L), jnp.float32),
            pltpu.VMEM((2, BB, C, L), jnp.float32),
            pltpu.VMEM((C2, INTER), jnp.float32),
            pltpu.VMEM((B, 1, L), jnp.float32),
            pltpu.VMEM((C2, 2), jnp.float32),
            pltpu.SemaphoreType.DMA((2,)),
            pltpu.SemaphoreType.DMA((2,)),
            pltpu.SemaphoreType.DMA((2,)),
            pltpu.SemaphoreType.DMA,
            pltpu.SemaphoreType.DMA,
            pltpu.SemaphoreType.DMA,
        ],
        name="adain_fused",
    )(u_i, x, e_qid, W, t, vb)


# speedup vs baseline: 3.2647x; 1.0305x over previous
import jax
import jax.numpy as jnp
from jax.experimental import pallas as pl
from jax.experimental.pallas import tpu as pltpu

B, C, L = 16, 256, 1024
DIM_U, Q_EMB = 64, 64
INTER = DIM_U * Q_EMB
C2 = 2 * C
EPS = 1e-5
BB = 2
NCH = B // BB
DEPTH = 3


def _body(u_smem, x_any, e_any, w_any, t_any, vb_any, out_any,
          xbuf, ebuf, obuf, wbuf, tbuf, vbbuf,
          xsem, esem, osem, wsem, tsem, vbsem):
    def copy_in(i, slot):
        pltpu.make_async_copy(
            x_any.at[pl.ds(BB * i, BB)], xbuf.at[slot], xsem.at[slot]).start()
        pltpu.make_async_copy(
            e_any.at[pl.ds(BB * i, BB)], ebuf.at[slot], esem.at[slot]).start()

    pltpu.make_async_copy(w_any, wbuf, wsem).start()
    pltpu.make_async_copy(t_any, tbuf, tsem).start()
    pltpu.make_async_copy(vb_any, vbbuf, vbsem).start()
    copy_in(0, 0)
    copy_in(1, 1)
    pltpu.make_async_copy(w_any, wbuf, wsem).wait()
    pltpu.make_async_copy(t_any, tbuf, tsem).wait()
    pltpu.make_async_copy(vb_any, vbbuf, vbsem).wait()

    lane = jax.lax.broadcasted_iota(jnp.int32, (1, 128), 1)
    ones_row = jnp.ones((1, L), jnp.float32)
    for i in range(NCH):
        slot = i % DEPTH
        if i + 2 < NCH:
            copy_in(i + 2, (i + 2) % DEPTH)
        pltpu.make_async_copy(
            x_any.at[pl.ds(BB * i, BB)], xbuf.at[slot], xsem.at[slot]).wait()
        pltpu.make_async_copy(
            e_any.at[pl.ds(BB * i, BB)], ebuf.at[slot], esem.at[slot]).wait()
        if i >= DEPTH:
            pltpu.make_async_copy(
                obuf.at[slot], out_any.at[pl.ds(BB * (i - DEPTH), BB)],
                osem.at[slot]).wait()
        for j in range(BB):
            b = BB * i + j
            xb = xbuf[slot, j]
            mu = jnp.sum(xb, axis=1, keepdims=True) * (1.0 / L)
            var = jnp.sum(xb * xb, axis=1, keepdims=True) * (1.0 / L) - mu * mu
            rstd = jax.lax.rsqrt(var + EPS)
            acc = jnp.zeros((C2, 128), jnp.float32)
            for k in range(DIM_U // 2):
                s0 = u_smem[b, 2 * k]
                s1 = u_smem[b, 2 * k + 1]
                m = jnp.where(lane < Q_EMB, s0, s1)
                acc = acc + wbuf[:, 128 * k:128 * (k + 1)] * m
            lhs = jnp.concatenate([acc, vbbuf[...]], axis=1)
            rhs = jnp.concatenate(
                [ebuf[slot, j], ebuf[slot, j], tbuf[b], ones_row], axis=0)
            params = jnp.dot(lhs, rhs, preferred_element_type=jnp.float32)
            nx = (xb - mu) * rstd
            obuf[slot, j] = params[:C] * nx + params[C:]
        pltpu.make_async_copy(
            obuf.at[slot], out_any.at[pl.ds(BB * i, BB)], osem.at[slot]).start()
    for i in range(NCH - DEPTH, NCH):
        slot = i % DEPTH
        pltpu.make_async_copy(
            obuf.at[slot], out_any.at[pl.ds(BB * i, BB)], osem.at[slot]).wait()


def kernel(x, u_i, e_qid, t, W, V, bias):
    vb = jnp.concatenate(
        [V, bias[:, None] + (jnp.arange(C2) < C).astype(jnp.float32)[:, None]],
        axis=1)
    return pl.pallas_call(
        _body,
        out_shape=jax.ShapeDtypeStruct((B, C, L), jnp.float32),
        in_specs=[
            pl.BlockSpec(memory_space=pltpu.SMEM),
            pl.BlockSpec(memory_space=pl.ANY),
            pl.BlockSpec(memory_space=pl.ANY),
            pl.BlockSpec(memory_space=pl.ANY),
            pl.BlockSpec(memory_space=pl.ANY),
            pl.BlockSpec(memory_space=pl.ANY),
        ],
        out_specs=pl.BlockSpec(memory_space=pl.ANY),
        scratch_shapes=[
            pltpu.VMEM((DEPTH, BB, C, L), jnp.float32),
            pltpu.VMEM((DEPTH, BB, Q_EMB, L), jnp.float32),
            pltpu.VMEM((DEPTH, BB, C, L), jnp.float32),
            pltpu.VMEM((C2, INTER), jnp.float32),
            pltpu.VMEM((B, 1, L), jnp.float32),
            pltpu.VMEM((C2, 2), jnp.float32),
            pltpu.SemaphoreType.DMA((DEPTH,)),
            pltpu.SemaphoreType.DMA((DEPTH,)),
            pltpu.SemaphoreType.DMA((DEPTH,)),
            pltpu.SemaphoreType.DMA,
            pltpu.SemaphoreType.DMA,
            pltpu.SemaphoreType.DMA,
        ],
        name="adain_fused",
    )(u_i, x, e_qid, W, t, vb)
